# baseline (speedup 1.0000x reference)
import jax
import jax.numpy as jnp
from jax import lax
from jax.experimental import pallas as pl
from jax.experimental.pallas import tpu as pltpu
from jax.experimental.pallas import tpu_sc as plsc

HIDDEN = 128
PITCH = 137
SPITCH = 17
LANES = 16
T = 128
EPS = 1e-12


def _rsqrt(x):
    bits = plsc.bitcast(x, jnp.int32)
    y = plsc.bitcast(jnp.int32(0x5F3759DF) - (bits >> 1), jnp.float32)
    for _ in range(3):
        y = y * (1.5 - 0.5 * x * y * y)
    return y


def kernel(input_ids, token_type_ids, word_emb, pos_emb, type_emb, gamma, beta):
    B, S = input_ids.shape
    N = B * S
    ids = input_ids.reshape(N).astype(jnp.int32)
    tts = token_type_ids.reshape(N).astype(jnp.int32)
    ptable = (pos_emb[:S, None, :] + type_emb[None, :, :]).reshape(S * 2, HIDDEN)
    ptable = jnp.pad(ptable, ((0, 0), (0, PITCH - HIDDEN))).reshape(-1)

    info = plsc.get_sparse_core_info()
    NC, NS = info.num_cores, info.num_subcores
    NW = NC * NS
    ntok = N // NW
    nchunks = ntok // T

    mesh = plsc.VectorSubcoreMesh(core_axis_name="c", subcore_axis_name="s")

    idx_t = pltpu.VMEM((T,), jnp.int32)
    raw_t = pltpu.VMEM((T, HIDDEN), jnp.float32)

    @pl.kernel(
        mesh=mesh,
        compiler_params=pltpu.CompilerParams(needs_layout_passes=False),
        out_type=jax.ShapeDtypeStruct((N, HIDDEN), jnp.float32),
        scratch_types=[
            idx_t, idx_t, idx_t,
            idx_t, idx_t, idx_t,
            idx_t, idx_t, idx_t,
            raw_t, raw_t, raw_t,
            pltpu.VMEM((T * PITCH,), jnp.float32),
            pltpu.VMEM((2 * S * PITCH,), jnp.float32),
            pltpu.VMEM((HIDDEN,), jnp.float32),
            pltpu.VMEM((HIDDEN,), jnp.float32),
            pltpu.VMEM((T * SPITCH,), jnp.float32),
            pltpu.VMEM((T * SPITCH,), jnp.float32),
            pltpu.SemaphoreType.DMA,
            pltpu.SemaphoreType.DMA,
            pltpu.SemaphoreType.DMA,
        ],
    )
    def body(ids_h, tts_h, word_h, ptable_h, gamma_h, beta_h, out_h,
             widx0, widx1, widx2, tidx0, tidx1, tidx2, pidx0, pidx1, pidx2,
             raw0, raw1, raw2, buf, ptab, gam, bet, msp, rsp,
             sem_g, sem_o, sem_i):
        wid = lax.axis_index("s") * NC + lax.axis_index("c")
        widxs = (widx0, widx1, widx2)
        tidxs = (tidx0, tidx1, tidx2)
        pidxs = (pidx0, pidx1, pidx2)
        raws = (raw0, raw1, raw2)
        pltpu.sync_copy(ptable_h, ptab)
        pltpu.sync_copy(gamma_h, gam)
        pltpu.sync_copy(beta_h, bet)
        lanes = lax.iota(jnp.int32, 16)
        lanes_p = lanes * PITCH
        zeros = jnp.zeros((LANES,), jnp.float32)
        inv_h = jnp.float32(1.0 / HIDDEN)
        gks = [gam[pl.ds(k * LANES, LANES)] for k in range(HIDDEN // LANES)]
        bks = [bet[pl.ds(k * LANES, LANES)] for k in range(HIDDEN // LANES)]

        def issue_ids(c, widx, tidx):
            base = wid * ntok + c * T
            pltpu.async_copy(ids_h.at[pl.ds(base, T)], widx, sem_i)
            pltpu.async_copy(tts_h.at[pl.ds(base, T)], tidx, sem_i)

        def wait_ids(c, widx, tidx):
            base = wid * ntok + c * T
            pltpu.make_async_copy(ids_h.at[pl.ds(base, T)], widx, sem_i).wait()
            pltpu.make_async_copy(tts_h.at[pl.ds(base, T)], tidx, sem_i).wait()

        def build_pidx(c, tidx, pidx):
            base = wid * ntok + c * T

            def mk(i, _):
                tt = tidx[pl.ds(i * LANES, LANES)]
                pos = (base + i * LANES + lanes) % S
                pidx[pl.ds(i * LANES, LANES)] = (pos * 2 + tt) * PITCH
                return 0

            lax.fori_loop(0, T // LANES, mk, 0)

        def compute(c, raw, pidx):
            @plsc.parallel_loop(0, T, unroll=8)
            def repitch(t):
                for k in range(HIDDEN // LANES):
                    buf[pl.ds(t * PITCH + k * LANES, LANES)] = (
                        raw[t, pl.ds(k * LANES, LANES)])

            def tb_body(tb, _):
                tok_p = tb * (LANES * PITCH) + lanes_p
                pp = pidx[pl.ds(tb * LANES, LANES)]

                @plsc.parallel_loop(0, HIDDEN, unroll=16,
                                    carry=(zeros, zeros, tok_p, pp))
                def p1(h, carry):
                    s, q, ia, ib = carry
                    v = plsc.load_gather(buf, [ia])
                    v = v + plsc.load_gather(ptab, [ib])
                    plsc.store_scatter(buf, [ia], v)
                    return s + v, q + v * v, ia + 1, ib + 1

                s, q, _, _ = p1
                mean = s * inv_h
                var = q * inv_h - mean * mean
                rstd = _rsqrt(var + EPS)
                spb = tb * (LANES * SPITCH) + lanes * SPITCH
                for cc in range(LANES):
                    plsc.store_scatter(msp, [spb + cc], mean)
                    plsc.store_scatter(rsp, [spb + cc], rstd)
                return 0

            lax.fori_loop(0, T // LANES, tb_body, 0)

            @plsc.parallel_loop(0, T, unroll=8)
            def norm_out(t):
                m = msp[pl.ds(t * SPITCH, LANES)]
                r = rsp[pl.ds(t * SPITCH, LANES)]
                for k in range(HIDDEN // LANES):
                    v = buf[pl.ds(t * PITCH + k * LANES, LANES)]
                    raw[t, pl.ds(k * LANES, LANES)] = (v - m) * r * gks[k] + bks[k]

        def step(c, ri):
            raw_c, pidx_c, widx_c = raws[ri], pidxs[ri], widxs[ri]
            rn = (ri + 1) % 3
            rn2 = (ri + 2) % 3
            raw_n, widx_n, tidx_n, pidx_n = raws[rn], widxs[rn], tidxs[rn], pidxs[rn]
            base = wid * ntok + c * T

            @pl.when(c >= 2)
            def _():
                pltpu.make_async_copy(out_h.at[pl.ds(base, T)], raw_n,
                                      sem_o).wait()

            @pl.when(c + 1 < nchunks)
            def _():
                wait_ids(c + 1, widx_n, tidx_n)
                build_pidx(c + 1, tidx_n, pidx_n)
                pltpu.async_copy(word_h.at[widx_n], raw_n, sem_g)

            @pl.when(c + 2 < nchunks)
            def _():
                issue_ids(c + 2, widxs[rn2], tidxs[rn2])

            pltpu.make_async_copy(word_h.at[widx_c], raw_c, sem_g).wait()
            compute(c, raw_c, pidx_c)
            pltpu.async_copy(raw_c, out_h.at[pl.ds(base, T)], sem_o)

        issue_ids(0, widx0, tidx0)
        wait_ids(0, widx0, tidx0)
        build_pidx(0, tidx0, pidx0)
        pltpu.async_copy(word_h.at[widx0], raw0, sem_g)
        issue_ids(1, widx1, tidx1)

        def chunk_loop(c, _):
            rr = c % 3
            for ri in range(3):
                @pl.when(rr == ri)
                def _():
                    step(c, ri)
            return 0

        lax.fori_loop(0, nchunks, chunk_loop, 0)

        for c in (nchunks - 2, nchunks - 1):
            pltpu.make_async_copy(out_h.at[pl.ds(wid * ntok + c * T, T)],
                                  raws[c % 3], sem_o).wait()

    out = body(ids, tts, word_emb, ptable, gamma, beta)
    return out.reshape(B, S, HIDDEN)


# speedup vs baseline: 1.1676x; 1.1676x over previous
import jax
import jax.numpy as jnp
from jax import lax
from jax.experimental import pallas as pl
from jax.experimental.pallas import tpu as pltpu
from jax.experimental.pallas import tpu_sc as plsc

HIDDEN = 128
PITCH = 137
SPITCH = 17
LANES = 16
T = 128
EPS = 1e-12


def _rsqrt(x):
    bits = plsc.bitcast(x, jnp.int32)
    y = plsc.bitcast(jnp.int32(0x5F3759DF) - (bits >> 1), jnp.float32)
    for _ in range(3):
        y = y * (1.5 - 0.5 * x * y * y)
    return y


def kernel(input_ids, token_type_ids, word_emb, pos_emb, type_emb, gamma, beta):
    B, S = input_ids.shape
    N = B * S
    ids = input_ids.reshape(N).astype(jnp.int32)
    tts = token_type_ids.reshape(N).astype(jnp.int32)
    ptable = (pos_emb[:S, None, :] + type_emb[None, :, :]).reshape(S * 2, HIDDEN)
    ptable = jnp.pad(ptable, ((0, 0), (0, PITCH - HIDDEN))).reshape(-1)

    info = plsc.get_sparse_core_info()
    NC, NS = info.num_cores, info.num_subcores
    NW = NC * NS
    ntok = N // NW
    nchunks = ntok // T

    mesh = plsc.VectorSubcoreMesh(core_axis_name="c", subcore_axis_name="s")

    idx_t = pltpu.VMEM((T,), jnp.int32)
    raw_t = pltpu.VMEM((T, HIDDEN), jnp.float32)

    @pl.kernel(
        mesh=mesh,
        compiler_params=pltpu.CompilerParams(needs_layout_passes=False),
        out_type=jax.ShapeDtypeStruct((N, HIDDEN), jnp.float32),
        scratch_types=[
            idx_t, idx_t, idx_t,
            idx_t, idx_t, idx_t,
            idx_t, idx_t, idx_t,
            raw_t, raw_t, raw_t,
            pltpu.VMEM((2 * S * PITCH,), jnp.float32),
            pltpu.VMEM((HIDDEN,), jnp.float32),
            pltpu.VMEM((HIDDEN,), jnp.float32),
            pltpu.VMEM((T * SPITCH,), jnp.float32),
            pltpu.VMEM((T * SPITCH,), jnp.float32),
            pltpu.SemaphoreType.DMA,
            pltpu.SemaphoreType.DMA,
            pltpu.SemaphoreType.DMA,
        ],
    )
    def body(ids_h, tts_h, word_h, ptable_h, gamma_h, beta_h, out_h,
             widx0, widx1, widx2, tidx0, tidx1, tidx2, pidx0, pidx1, pidx2,
             raw0, raw1, raw2, ptab, gam, bet, msp, rsp,
             sem_g, sem_o, sem_i):
        wid = lax.axis_index("s") * NC + lax.axis_index("c")
        widxs = (widx0, widx1, widx2)
        tidxs = (tidx0, tidx1, tidx2)
        pidxs = (pidx0, pidx1, pidx2)
        raws = (raw0, raw1, raw2)
        pltpu.sync_copy(ptable_h, ptab)
        pltpu.sync_copy(gamma_h, gam)
        pltpu.sync_copy(beta_h, bet)
        lanes = lax.iota(jnp.int32, 16)
        lanes_p = lanes * PITCH
        zeros = jnp.zeros((LANES,), jnp.float32)
        inv_h = jnp.float32(1.0 / HIDDEN)
        gks = [gam[pl.ds(k * LANES, LANES)] for k in range(HIDDEN // LANES)]
        bks = [bet[pl.ds(k * LANES, LANES)] for k in range(HIDDEN // LANES)]

        def issue_ids(c, widx, tidx):
            base = wid * ntok + c * T
            pltpu.async_copy(ids_h.at[pl.ds(base, T)], widx, sem_i)
            pltpu.async_copy(tts_h.at[pl.ds(base, T)], tidx, sem_i)

        def wait_ids(c, widx, tidx):
            base = wid * ntok + c * T
            pltpu.make_async_copy(ids_h.at[pl.ds(base, T)], widx, sem_i).wait()
            pltpu.make_async_copy(tts_h.at[pl.ds(base, T)], tidx, sem_i).wait()

        def build_pidx(c, tidx, pidx):
            base = wid * ntok + c * T

            def mk(i, _):
                tt = tidx[pl.ds(i * LANES, LANES)]
                pos = (base + i * LANES + lanes) % S
                pidx[pl.ds(i * LANES, LANES)] = (pos * 2 + tt) * PITCH
                return 0

            lax.fori_loop(0, T // LANES, mk, 0)

        def compute(c, raw, pidx):
            def tb_body(tb, _):
                tok = tb * LANES + lanes
                pp = pidx[pl.ds(tb * LANES, LANES)]

                @plsc.parallel_loop(0, HIDDEN, unroll=8,
                                    carry=(zeros, zeros, lanes))
                def p1(h, carry):
                    s, q, e = carry
                    v = plsc.load_gather(raw, [tok, e])
                    v = v + plsc.load_gather(ptab, [pp + e])
                    plsc.store_scatter(raw, [tok, e], v)
                    return s + v, q + v * v, (e + 1) & (HIDDEN - 1)

                s, q, _ = p1
                mean = s * inv_h
                var = q * inv_h - mean * mean
                rstd = _rsqrt(var + EPS)
                spb = tb * (LANES * SPITCH) + lanes * SPITCH
                for cc in range(LANES):
                    plsc.store_scatter(msp, [spb + cc], mean)
                    plsc.store_scatter(rsp, [spb + cc], rstd)
                return 0

            lax.fori_loop(0, T // LANES, tb_body, 0)

            @plsc.parallel_loop(0, T, unroll=4)
            def norm_out(t):
                m = msp[pl.ds(t * SPITCH, LANES)]
                r = rsp[pl.ds(t * SPITCH, LANES)]
                for k in range(HIDDEN // LANES):
                    v = raw[t, pl.ds(k * LANES, LANES)]
                    raw[t, pl.ds(k * LANES, LANES)] = (v - m) * r * gks[k] + bks[k]

        def step(c, ri):
            raw_c, pidx_c, widx_c = raws[ri], pidxs[ri], widxs[ri]
            rn = (ri + 1) % 3
            rn2 = (ri + 2) % 3
            raw_n, widx_n, tidx_n, pidx_n = raws[rn], widxs[rn], tidxs[rn], pidxs[rn]
            base = wid * ntok + c * T

            @pl.when(c >= 2)
            def _():
                pltpu.make_async_copy(out_h.at[pl.ds(base, T)], raw_n,
                                      sem_o).wait()

            @pl.when(c + 1 < nchunks)
            def _():
                wait_ids(c + 1, widx_n, tidx_n)
                build_pidx(c + 1, tidx_n, pidx_n)
                pltpu.async_copy(word_h.at[widx_n], raw_n, sem_g)

            @pl.when(c + 2 < nchunks)
            def _():
                issue_ids(c + 2, widxs[rn2], tidxs[rn2])

            pltpu.make_async_copy(word_h.at[widx_c], raw_c, sem_g).wait()
            compute(c, raw_c, pidx_c)
            pltpu.async_copy(raw_c, out_h.at[pl.ds(base, T)], sem_o)

        issue_ids(0, widx0, tidx0)
        wait_ids(0, widx0, tidx0)
        build_pidx(0, tidx0, pidx0)
        pltpu.async_copy(word_h.at[widx0], raw0, sem_g)
        issue_ids(1, widx1, tidx1)

        def chunk_loop(c, _):
            rr = c % 3
            for ri in range(3):
                @pl.when(rr == ri)
                def _():
                    step(c, ri)
            return 0

        lax.fori_loop(0, nchunks, chunk_loop, 0)

        for c in (nchunks - 2, nchunks - 1):
            pltpu.make_async_copy(out_h.at[pl.ds(wid * ntok + c * T, T)],
                                  raws[c % 3], sem_o).wait()

    out = body(ids, tts, word_emb, ptable, gamma, beta)
    return out.reshape(B, S, HIDDEN)
